# Initial kernel scaffold; baseline (speedup 1.0000x reference)
import jax
import jax.numpy as jnp
from jax.experimental import pallas as pl


def kernel(z, pos, batch, emb, mlp_w1, mlp_b1, mlp_w2, mlp_b2, conv_w1, conv_w2, conv_b2, lin_w, lin_b, out_w1, out_b1, out_w2, out_b2):
    raise NotImplementedError("write your pallas kernel here")


# speedup vs baseline: 7.3418x; 7.3418x over previous
import functools
import math

import jax
import jax.numpy as jnp
from jax import lax
from jax.experimental import pallas as pl
from jax.experimental.pallas import tpu as pltpu
from jax.experimental.pallas import tpu_sc as plsc

N = 10000
H = 128
NF = 128
NG = 50
NGP = 64
NI = 6
K = 32
CUTOFF = 10.0
NGRAPH = 100

BR = 128
BC = 128
NP = 10240
NB = NP // BR
E = NP * K

LOG2 = math.log(2.0)
DELTA = CUTOFF / (NG - 1)
GCOEFF = -0.5 / (DELTA * DELTA)

NC_SC = 2
NS_SC = 16
NWORK = NC_SC * NS_SC
EPW = E // NWORK
CH = 128
NCH = EPW // CH


def _ssp(x):
    return jnp.maximum(x, 0.0) + jnp.log1p(jnp.exp(-jnp.abs(x))) - LOG2


def _nbr_body(cs_ref, nt_ref, rowf_ref, colf_ref, rs_ref, re_ref, rid_ref,
              src_ref, w_ref, c_ref):
    b = pl.program_id(0)
    n_tiles = nt_ref[b]
    cs_tile = cs_ref[b]

    rowf = rowf_ref[...]
    rs = rs_ref[...]
    re = re_ref[...]
    rid = rid_ref[...]

    inf = jnp.float32(jnp.inf)
    best_d0 = jnp.full((K, BR), inf, jnp.float32)
    best_i0 = jnp.zeros((K, BR), jnp.int32)
    krow = lax.broadcasted_iota(jnp.int32, (K, BR), 0)

    def tile_body(ct, carry):
        bd, bi = carry
        c0 = (cs_tile + ct) * BC
        colf = colf_ref[pl.ds(c0, BC), :]
        d2 = lax.dot_general(colf, rowf, (((1,), (1,)), ((), ())),
                             preferred_element_type=jnp.float32)
        cg = c0 + lax.broadcasted_iota(jnp.int32, (BC, 1), 0)
        m = (cg >= rs) & (cg < re) & (cg != rid)
        d2m = jnp.where(m, jnp.maximum(d2, 0.0), inf)
        cgb = jnp.broadcast_to(cg, (BC, BR))

        comb_d = jnp.concatenate([bd, d2m], axis=0)
        comb_i = jnp.concatenate([bi, cgb], axis=0)
        nd = jnp.full((K, BR), inf, jnp.float32)
        ni = jnp.zeros((K, BR), jnp.int32)
        big = jnp.int32(1 << 30)
        for k in range(K):
            mv = jnp.min(comb_d, axis=0, keepdims=True)
            ismin = comb_d == mv
            iv = jnp.min(jnp.where(ismin, comb_i, big), axis=0,
                         keepdims=True)
            sel = ismin & (comb_i == iv)
            nd = jnp.where(krow == k, mv, nd)
            ni = jnp.where(krow == k, iv, ni)
            comb_d = jnp.where(sel, inf, comb_d)
        return nd, ni

    best_d, best_i = lax.fori_loop(0, n_tiles, tile_body, (best_d0, best_i0))

    valid = best_d < jnp.float32(CUTOFF * CUTOFF)
    ridb = jnp.broadcast_to(rid, (K, BR))
    src = jnp.where(valid, best_i, ridb)
    d2e = jnp.where(valid, jnp.maximum(best_d, 0.0),
                    jnp.float32(CUTOFF * CUTOFF))
    wv = jnp.sqrt(d2e)
    cv = 0.5 * (jnp.cos(wv * jnp.float32(math.pi / CUTOFF)) + 1.0)
    src_ref[...] = src
    w_ref[...] = wv
    c_ref[...] = cv


def _neighbor_search(cs_tile, n_tiles, rowfeat, colfeat, rsT, reT, ridT):
    grid_spec = pltpu.PrefetchScalarGridSpec(
        num_scalar_prefetch=2,
        grid=(NB,),
        in_specs=[
            pl.BlockSpec((BR, 8), lambda b, cs, nt: (b, 0)),
            pl.BlockSpec((NP, 8), lambda b, cs, nt: (0, 0)),
            pl.BlockSpec((1, BR), lambda b, cs, nt: (0, b)),
            pl.BlockSpec((1, BR), lambda b, cs, nt: (0, b)),
            pl.BlockSpec((1, BR), lambda b, cs, nt: (0, b)),
        ],
        out_specs=[
            pl.BlockSpec((K, BR), lambda b, cs, nt: (b, 0)),
            pl.BlockSpec((K, BR), lambda b, cs, nt: (b, 0)),
            pl.BlockSpec((K, BR), lambda b, cs, nt: (b, 0)),
        ],
    )
    return pl.pallas_call(
        _nbr_body,
        grid_spec=grid_spec,
        out_shape=[
            jax.ShapeDtypeStruct((NB * K, BR), jnp.int32),
            jax.ShapeDtypeStruct((NB * K, BR), jnp.float32),
            jax.ShapeDtypeStruct((NB * K, BR), jnp.float32),
        ],
    )(cs_tile, n_tiles, rowfeat, colfeat, rsT, reT, ridT)


def _embed_body(z_ref, emb_ref, cw1_ref, h_ref, xh_ref):
    zt = z_ref[...]
    lane = lax.broadcasted_iota(jnp.int32, (BR, 128), 1)
    onehot = (lane == zt).astype(jnp.float32)
    h = jnp.dot(onehot, emb_ref[...],
                preferred_element_type=jnp.float32)
    h_ref[...] = h
    xh_ref[...] = jnp.dot(h, cw1_ref[...], preferred_element_type=jnp.float32)


def _embed(z_pad, emb_pad, cw1_0):
    return pl.pallas_call(
        _embed_body,
        grid=(NB,),
        in_specs=[
            pl.BlockSpec((BR, 1), lambda b: (b, 0)),
            pl.BlockSpec((128, H), lambda b: (0, 0)),
            pl.BlockSpec((H, H), lambda b: (0, 0)),
        ],
        out_specs=[
            pl.BlockSpec((BR, H), lambda b: (b, 0)),
            pl.BlockSpec((BR, H), lambda b: (b, 0)),
        ],
        out_shape=[
            jax.ShapeDtypeStruct((NP, H), jnp.float32),
            jax.ShapeDtypeStruct((NP, H), jnp.float32),
        ],
    )(z_pad, emb_pad, cw1_0)


def _sc_gather(xh, src):
    mesh = plsc.VectorSubcoreMesh(core_axis_name="c", subcore_axis_name="s")

    @functools.partial(
        pl.kernel,
        mesh=mesh,
        out_type=jax.ShapeDtypeStruct((E, H), jnp.float32),
        scratch_types=[
            pltpu.VMEM((CH,), jnp.int32),
            pltpu.VMEM((CH, H), jnp.float32),
            pltpu.SemaphoreType.DMA,
        ],
    )
    def gather_k(xh_hbm, src_hbm, out_hbm, idx_v, rows_v, sem):
        wid = lax.axis_index("s") * NC_SC + lax.axis_index("c")

        def body(ch, carry):
            base = wid * EPW + ch * CH
            pltpu.sync_copy(src_hbm.at[pl.ds(base, CH)], idx_v)
            pltpu.async_copy(xh_hbm.at[idx_v], rows_v, sem).wait()
            pltpu.sync_copy(rows_v, out_hbm.at[pl.ds(base, CH)])
            return carry

        lax.fori_loop(0, NCH, body, 0)

    return gather_k(xh, src)


def _layer_body(w_ref, c_ref, g_ref, h_ref, w1_ref, b1_ref, w2_ref, b2_ref,
                cw2_ref, cb2_ref, lw_ref, lb_ref, cw1n_ref,
                hn_ref, xhn_ref):
    wt = w_ref[...]
    offs = lax.broadcasted_iota(jnp.int32, (1, NGP), 1).astype(
        jnp.float32) * jnp.float32(DELTA)
    d = wt - offs
    attr = jnp.exp(jnp.float32(GCOEFF) * d * d)
    z1 = jnp.dot(attr, w1_ref[...], preferred_element_type=jnp.float32)
    z1 = _ssp(z1 + b1_ref[...])
    z2 = jnp.dot(z1, w2_ref[...], preferred_element_type=jnp.float32)
    z2 = (z2 + b2_ref[...]) * c_ref[...]
    msg = g_ref[...] * z2
    m3 = msg.reshape(BR, K, H)
    agg = jnp.sum(m3, axis=1)
    t = jnp.dot(agg, cw2_ref[...], preferred_element_type=jnp.float32)
    t = _ssp(t + cb2_ref[...])
    u = jnp.dot(t, lw_ref[...], preferred_element_type=jnp.float32)
    hn = h_ref[...] + u + lb_ref[...]
    hn_ref[...] = hn
    xhn_ref[...] = jnp.dot(hn, cw1n_ref[...],
                           preferred_element_type=jnp.float32)


def _layer(wcol, ccol, gath, h, w1p, b1, w2, b2, cw2, cb2, lw, lb, cw1n):
    EB = BR * K
    return pl.pallas_call(
        _layer_body,
        grid=(NB,),
        in_specs=[
            pl.BlockSpec((EB, 1), lambda b: (b, 0)),
            pl.BlockSpec((EB, 1), lambda b: (b, 0)),
            pl.BlockSpec((EB, H), lambda b: (b, 0)),
            pl.BlockSpec((BR, H), lambda b: (b, 0)),
            pl.BlockSpec((NGP, NF), lambda b: (0, 0)),
            pl.BlockSpec((1, NF), lambda b: (0, 0)),
            pl.BlockSpec((NF, NF), lambda b: (0, 0)),
            pl.BlockSpec((1, NF), lambda b: (0, 0)),
            pl.BlockSpec((NF, H), lambda b: (0, 0)),
            pl.BlockSpec((1, H), lambda b: (0, 0)),
            pl.BlockSpec((H, H), lambda b: (0, 0)),
            pl.BlockSpec((1, H), lambda b: (0, 0)),
            pl.BlockSpec((H, H), lambda b: (0, 0)),
        ],
        out_specs=[
            pl.BlockSpec((BR, H), lambda b: (b, 0)),
            pl.BlockSpec((BR, H), lambda b: (b, 0)),
        ],
        out_shape=[
            jax.ShapeDtypeStruct((NP, H), jnp.float32),
            jax.ShapeDtypeStruct((NP, H), jnp.float32),
        ],
    )(wcol, ccol, gath, h, w1p, b1, w2, b2, cw2, cb2, lw, lb, cw1n)


def _readout_body(h_ref, bt_ref, ow1_ref, ob1_ref, ow2_ref, ob2_ref, out_ref):
    b = pl.program_id(0)
    h = h_ref[...]
    t = _ssp(jnp.dot(h, ow1_ref[...], preferred_element_type=jnp.float32)
             + ob1_ref[...])
    y = jnp.dot(t, ow2_ref[...], preferred_element_type=jnp.float32)
    y = y[:, 0:1] + ob2_ref[0, 0]
    gi = lax.broadcasted_iota(jnp.int32, (128, 1), 0)
    mask = (gi == bt_ref[...]).astype(jnp.float32)
    contrib = jnp.dot(mask, y, preferred_element_type=jnp.float32)

    @pl.when(b == 0)
    def _():
        out_ref[...] = jnp.zeros_like(out_ref)

    out_ref[...] += contrib


def _readout(h, batchT, ow1, ob1, ow2p, ob2):
    RB = 512
    return pl.pallas_call(
        _readout_body,
        grid=(NP // RB,),
        in_specs=[
            pl.BlockSpec((RB, H), lambda b: (b, 0)),
            pl.BlockSpec((1, RB), lambda b: (0, b)),
            pl.BlockSpec((H, NGP), lambda b: (0, 0)),
            pl.BlockSpec((1, NGP), lambda b: (0, 0)),
            pl.BlockSpec((NGP, 128), lambda b: (0, 0)),
            pl.BlockSpec((1, 1), lambda b: (0, 0), memory_space=pltpu.SMEM),
        ],
        out_specs=pl.BlockSpec((128, 1), lambda b: (0, 0)),
        out_shape=jax.ShapeDtypeStruct((128, 1), jnp.float32),
    )(h, batchT, ow1, ob1, ow2p, ob2)


def kernel(z, pos, batch, emb, mlp_w1, mlp_b1, mlp_w2, mlp_b2, conv_w1,
           conv_w2, conv_b2, lin_w, lin_b, out_w1, out_b1, out_w2, out_b2):
    zi = z.astype(jnp.int32)
    bi = batch.astype(jnp.int32)
    pos = pos.astype(jnp.float32)

    gids = jnp.arange(NGRAPH, dtype=jnp.int32)
    seg_start = jnp.searchsorted(bi, gids, side="left").astype(jnp.int32)
    seg_end = jnp.searchsorted(bi, gids, side="right").astype(jnp.int32)
    row_start = seg_start[bi]
    row_end = seg_end[bi]
    zpad = jnp.zeros((NP - N,), jnp.int32)
    row_start_p = jnp.concatenate([row_start, zpad])
    row_end_p = jnp.concatenate([row_end, zpad])

    pos_p = jnp.concatenate([pos, jnp.zeros((NP - N, 3), jnp.float32)], 0)
    sq = jnp.sum(pos_p * pos_p, axis=1, keepdims=True)
    ones = jnp.ones((NP, 1), jnp.float32)
    zeros3 = jnp.zeros((NP, 3), jnp.float32)
    rowfeat = jnp.concatenate([-2.0 * pos_p, sq, ones, zeros3], axis=1)
    colfeat = jnp.concatenate([pos_p, ones, sq, zeros3], axis=1)

    blo = row_start_p.reshape(NB, BR)[:, 0]
    bhi = jnp.max(row_end_p.reshape(NB, BR), axis=1)
    cs_tile = blo // BC
    n_tiles = (bhi + BC - 1) // BC - cs_tile

    rsT = row_start_p[None, :]
    reT = row_end_p[None, :]
    ridT = jnp.arange(NP, dtype=jnp.int32)[None, :]

    srcT, wT, cT = _neighbor_search(cs_tile, n_tiles, rowfeat, colfeat,
                                    rsT, reT, ridT)
    src = srcT.reshape(NB, K, BR).transpose(0, 2, 1).reshape(E)
    wcol = wT.reshape(NB, K, BR).transpose(0, 2, 1).reshape(E, 1)
    ccol = cT.reshape(NB, K, BR).transpose(0, 2, 1).reshape(E, 1)

    z_pad = jnp.concatenate([zi, jnp.zeros((NP - N,), jnp.int32)])[:, None]
    emb_pad = jnp.concatenate(
        [emb.astype(jnp.float32), jnp.zeros((28, H), jnp.float32)], 0)
    w1p = jnp.concatenate(
        [mlp_w1, jnp.zeros((NI, NGP - NG, NF), mlp_w1.dtype)], 1)
    ow1 = jnp.concatenate(
        [out_w1, jnp.zeros((H, NGP - H // 2), out_w1.dtype)], 1)
    ob1 = jnp.concatenate(
        [out_b1, jnp.zeros((NGP - H // 2,), out_b1.dtype)], 0)[None, :]
    ow2p = jnp.zeros((NGP, 128), jnp.float32).at[: H // 2, 0].set(out_w2[:, 0])
    ob2 = out_b2.reshape(1, 1)
    batchT = jnp.concatenate(
        [bi, jnp.full((NP - N,), 127, jnp.int32)])[None, :]

    h, xh = _embed(z_pad, emb_pad, conv_w1[0])

    zeros_hh = jnp.zeros((H, H), jnp.float32)
    for i in range(NI):
        gath = _sc_gather(xh, src)
        cw1n = conv_w1[i + 1] if i + 1 < NI else zeros_hh
        h, xh = _layer(wcol, ccol, gath, h,
                       w1p[i], mlp_b1[i][None, :], mlp_w2[i],
                       mlp_b2[i][None, :], conv_w2[i], conv_b2[i][None, :],
                       lin_w[i], lin_b[i][None, :], cw1n)

    res = _readout(h, batchT, ow1, ob1, ow2p, ob2)
    return res[:NGRAPH]


# speedup vs baseline: 9.2144x; 1.2551x over previous
import functools
import math

import jax
import jax.numpy as jnp
from jax import lax
from jax.experimental import pallas as pl
from jax.experimental.pallas import tpu as pltpu
from jax.experimental.pallas import tpu_sc as plsc

N = 10000
H = 128
NF = 128
NG = 50
NGP = 64
NI = 6
K = 32
CUTOFF = 10.0
NGRAPH = 100

BR = 128
BC = 128
NP = 10240
NB = NP // BR
E = NP * K

LOG2 = math.log(2.0)
DELTA = CUTOFF / (NG - 1)
GCOEFF = -0.5 / (DELTA * DELTA)

NC_SC = 2
NS_SC = 16
NWORK = NC_SC * NS_SC
EPW = E // NWORK
CH = 128
NCH = EPW // CH


def _ssp(x):
    return jnp.maximum(x, 0.0) + jnp.log1p(jnp.exp(-jnp.abs(x))) - LOG2


def _nbr_body(cs_ref, nt_ref, rowt_ref, colf_ref, rs_ref, re_ref, rid_ref,
              src_ref, w_ref, c_ref):
    b = pl.program_id(0)
    n_tiles = nt_ref[b]
    cs_tile = cs_ref[b]

    rowt = rowt_ref[...]
    subl = lax.broadcasted_iota(jnp.int32, (8, 1), 0)
    rowp = jnp.where(subl < 3, rowt, 0.0)
    rx = rowt[0:1, :]
    ry = rowt[1:2, :]
    rz = rowt[2:3, :]
    sqr = rowt[3:4, :]
    rs = rs_ref[...]
    re = re_ref[...]
    rid = rid_ref[...]

    inf = jnp.float32(jnp.inf)
    best_d0 = jnp.full((K, BR), inf, jnp.float32)
    best_w0 = jnp.full((K, BR), inf, jnp.float32)
    best_i0 = jnp.zeros((K, BR), jnp.int32)
    krow = lax.broadcasted_iota(jnp.int32, (K, BR), 0)

    def tile_body(ct, carry):
        bd, bw, bi = carry
        c0 = (cs_tile + ct) * BC
        colf = colf_ref[pl.ds(c0, BC), :]
        sqc = colf[:, 3:4]
        lane8 = lax.broadcasted_iota(jnp.int32, (1, 8), 1)
        colp = jnp.where(lane8 < 3, colf, 0.0)
        dot = lax.dot_general(colp, rowp, (((1,), (0,)), ((), ())),
                              preferred_element_type=jnp.float32)
        d2 = (sqc + sqr) - 2.0 * dot
        xc = colf[:, 0:1]
        yc = colf[:, 1:2]
        zc = colf[:, 2:3]
        dotf = (xc * rx + yc * ry) + zc * rz
        d2w = jnp.maximum((sqc + sqr) - 2.0 * dotf, 0.0)
        cg = c0 + lax.broadcasted_iota(jnp.int32, (BC, 1), 0)
        m = (cg >= rs) & (cg < re) & (cg != rid)
        d2m = jnp.where(m, jnp.maximum(d2, 0.0), inf)
        cgb = jnp.broadcast_to(cg, (BC, BR))

        comb_d = jnp.concatenate([bd, d2m], axis=0)
        comb_w = jnp.concatenate([bw, jnp.broadcast_to(d2w, (BC, BR))], axis=0)
        comb_i = jnp.concatenate([bi, cgb], axis=0)
        nd = jnp.full((K, BR), inf, jnp.float32)
        nw = jnp.full((K, BR), inf, jnp.float32)
        ni = jnp.zeros((K, BR), jnp.int32)
        big = jnp.int32(1 << 30)
        for k in range(K):
            mv = jnp.min(comb_d, axis=0, keepdims=True)
            ismin = comb_d == mv
            iv = jnp.min(jnp.where(ismin, comb_i, big), axis=0,
                         keepdims=True)
            sel = ismin & (comb_i == iv)
            vw = jnp.min(jnp.where(sel, comb_w, inf), axis=0,
                         keepdims=True)
            nd = jnp.where(krow == k, mv, nd)
            nw = jnp.where(krow == k, vw, nw)
            ni = jnp.where(krow == k, iv, ni)
            comb_d = jnp.where(sel, inf, comb_d)
        return nd, nw, ni

    best_d, best_w, best_i = lax.fori_loop(
        0, n_tiles, tile_body, (best_d0, best_w0, best_i0))

    valid = best_d < jnp.float32(CUTOFF * CUTOFF)
    ridb = jnp.broadcast_to(rid, (K, BR))
    src = jnp.where(valid, best_i, ridb)
    d2e = jnp.where(valid, best_w, jnp.float32(CUTOFF * CUTOFF))
    wv = jnp.sqrt(d2e)
    cv = 0.5 * (jnp.cos(wv * jnp.float32(math.pi / CUTOFF)) + 1.0)
    src_ref[...] = src
    w_ref[...] = wv
    c_ref[...] = cv


def _neighbor_search(cs_tile, n_tiles, rowT, colfeat, rsT, reT, ridT):
    grid_spec = pltpu.PrefetchScalarGridSpec(
        num_scalar_prefetch=2,
        grid=(NB,),
        in_specs=[
            pl.BlockSpec((8, BR), lambda b, cs, nt: (0, b)),
            pl.BlockSpec((NP, 8), lambda b, cs, nt: (0, 0)),
            pl.BlockSpec((1, BR), lambda b, cs, nt: (0, b)),
            pl.BlockSpec((1, BR), lambda b, cs, nt: (0, b)),
            pl.BlockSpec((1, BR), lambda b, cs, nt: (0, b)),
        ],
        out_specs=[
            pl.BlockSpec((K, BR), lambda b, cs, nt: (b, 0)),
            pl.BlockSpec((K, BR), lambda b, cs, nt: (b, 0)),
            pl.BlockSpec((K, BR), lambda b, cs, nt: (b, 0)),
        ],
    )
    return pl.pallas_call(
        _nbr_body,
        grid_spec=grid_spec,
        out_shape=[
            jax.ShapeDtypeStruct((NB * K, BR), jnp.int32),
            jax.ShapeDtypeStruct((NB * K, BR), jnp.float32),
            jax.ShapeDtypeStruct((NB * K, BR), jnp.float32),
        ],
    )(cs_tile, n_tiles, rowT, colfeat, rsT, reT, ridT)


def _embed_body(z_ref, emb_ref, cw1_ref, h_ref, xh_ref):
    zt = z_ref[...]
    lane = lax.broadcasted_iota(jnp.int32, (BR, 128), 1)
    onehot = (lane == zt).astype(jnp.float32)
    h = jnp.dot(onehot, emb_ref[...], preferred_element_type=jnp.float32,
                precision=lax.Precision.HIGHEST)
    h_ref[...] = h
    xh_ref[...] = jnp.dot(h, cw1_ref[...], preferred_element_type=jnp.float32)


def _embed(z_pad, emb_pad, cw1_0):
    return pl.pallas_call(
        _embed_body,
        grid=(NB,),
        in_specs=[
            pl.BlockSpec((BR, 1), lambda b: (b, 0)),
            pl.BlockSpec((128, H), lambda b: (0, 0)),
            pl.BlockSpec((H, H), lambda b: (0, 0)),
        ],
        out_specs=[
            pl.BlockSpec((BR, H), lambda b: (b, 0)),
            pl.BlockSpec((BR, H), lambda b: (b, 0)),
        ],
        out_shape=[
            jax.ShapeDtypeStruct((NP, H), jnp.float32),
            jax.ShapeDtypeStruct((NP, H), jnp.float32),
        ],
    )(z_pad, emb_pad, cw1_0)


NBUF = 4
NROUNDS = NCH // NBUF


def _sc_gather(xh, src2d):
    mesh = plsc.VectorSubcoreMesh(core_axis_name="c", subcore_axis_name="s")

    @functools.partial(
        pl.kernel,
        mesh=mesh,
        out_type=jax.ShapeDtypeStruct((E, H), jnp.float32),
        scratch_types=[
            pltpu.VMEM((NCH, CH), jnp.int32),
            pltpu.VMEM((NBUF, CH, H), jnp.float32),
        ]
        + [pltpu.SemaphoreType.DMA] * (2 * NBUF),
    )
    def gather_k(xh_hbm, src_hbm, out_hbm, idx_v, rows_v, *sems):
        gsem = sems[:NBUF]
        wsem = sems[NBUF:]
        wid = lax.axis_index("s") * NC_SC + lax.axis_index("c")
        ebase = wid * EPW
        pltpu.sync_copy(src_hbm.at[pl.ds(wid * NCH, NCH)], idx_v)

        def gstart(ch, b):
            pltpu.async_copy(xh_hbm.at[idx_v.at[ch]], rows_v.at[b], gsem[b])

        def gwait(b):
            pltpu.make_async_copy(xh_hbm.at[idx_v.at[0]], rows_v.at[b],
                                  gsem[b]).wait()

        def wstart(ch, b):
            pltpu.async_copy(rows_v.at[b],
                             out_hbm.at[pl.ds(ebase + ch * CH, CH)], wsem[b])

        def wwait(b):
            pltpu.make_async_copy(rows_v.at[b],
                                  out_hbm.at[pl.ds(ebase, CH)], wsem[b]).wait()

        for b in range(NBUF):
            gstart(b, b)

        def round_body(g, carry):
            for b in range(NBUF):
                gwait(b)
                wstart(g * NBUF + b, b)
            nxt = g + 1

            @pl.when(nxt < NROUNDS)
            def _():
                for b in range(NBUF):
                    wwait(b)
                    gstart(nxt * NBUF + b, b)

            return carry

        lax.fori_loop(0, NROUNDS, round_body, 0)
        for b in range(NBUF):
            wwait(b)

    return gather_k(xh, src2d)


def _layer_body(w_ref, c_ref, g_ref, h_ref, w1_ref, b1_ref, w2_ref, b2_ref,
                cw2_ref, cb2_ref, lw_ref, lb_ref, cw1n_ref,
                hn_ref, xhn_ref):
    wt = w_ref[...]
    offs = lax.broadcasted_iota(jnp.int32, (1, NGP), 1).astype(
        jnp.float32) * jnp.float32(DELTA)
    d = wt - offs
    attr = jnp.exp(jnp.float32(GCOEFF) * d * d)
    z1 = jnp.dot(attr, w1_ref[...], preferred_element_type=jnp.float32)
    z1 = _ssp(z1 + b1_ref[...])
    z2 = jnp.dot(z1, w2_ref[...], preferred_element_type=jnp.float32)
    z2 = (z2 + b2_ref[...]) * c_ref[...]
    msg = g_ref[...] * z2
    m3 = msg.reshape(BR, K, H)
    agg = jnp.sum(m3, axis=1)
    t = jnp.dot(agg, cw2_ref[...], preferred_element_type=jnp.float32)
    t = _ssp(t + cb2_ref[...])
    u = jnp.dot(t, lw_ref[...], preferred_element_type=jnp.float32)
    hn = h_ref[...] + u + lb_ref[...]
    hn_ref[...] = hn
    xhn_ref[...] = jnp.dot(hn, cw1n_ref[...],
                           preferred_element_type=jnp.float32)


def _layer(wcol, ccol, gath, h, w1p, b1, w2, b2, cw2, cb2, lw, lb, cw1n):
    EB = BR * K
    return pl.pallas_call(
        _layer_body,
        grid=(NB,),
        in_specs=[
            pl.BlockSpec((EB, 1), lambda b: (b, 0)),
            pl.BlockSpec((EB, 1), lambda b: (b, 0)),
            pl.BlockSpec((EB, H), lambda b: (b, 0)),
            pl.BlockSpec((BR, H), lambda b: (b, 0)),
            pl.BlockSpec((NGP, NF), lambda b: (0, 0)),
            pl.BlockSpec((1, NF), lambda b: (0, 0)),
            pl.BlockSpec((NF, NF), lambda b: (0, 0)),
            pl.BlockSpec((1, NF), lambda b: (0, 0)),
            pl.BlockSpec((NF, H), lambda b: (0, 0)),
            pl.BlockSpec((1, H), lambda b: (0, 0)),
            pl.BlockSpec((H, H), lambda b: (0, 0)),
            pl.BlockSpec((1, H), lambda b: (0, 0)),
            pl.BlockSpec((H, H), lambda b: (0, 0)),
        ],
        out_specs=[
            pl.BlockSpec((BR, H), lambda b: (b, 0)),
            pl.BlockSpec((BR, H), lambda b: (b, 0)),
        ],
        out_shape=[
            jax.ShapeDtypeStruct((NP, H), jnp.float32),
            jax.ShapeDtypeStruct((NP, H), jnp.float32),
        ],
    )(wcol, ccol, gath, h, w1p, b1, w2, b2, cw2, cb2, lw, lb, cw1n)


def _readout_body(h_ref, bt_ref, ow1_ref, ob1_ref, ow2_ref, ob2_ref, out_ref):
    b = pl.program_id(0)
    h = h_ref[...]
    t = _ssp(jnp.dot(h, ow1_ref[...], preferred_element_type=jnp.float32)
             + ob1_ref[...])
    y = jnp.dot(t, ow2_ref[...], preferred_element_type=jnp.float32)
    y = y[:, 0:1] + ob2_ref[0, 0]
    gi = lax.broadcasted_iota(jnp.int32, (128, 1), 0)
    mask = (gi == bt_ref[...]).astype(jnp.float32)
    contrib = jnp.dot(mask, y, preferred_element_type=jnp.float32,
                      precision=lax.Precision.HIGHEST)

    @pl.when(b == 0)
    def _():
        out_ref[...] = jnp.zeros_like(out_ref)

    out_ref[...] += contrib


def _readout(h, batchT, ow1, ob1, ow2p, ob2):
    RB = 512
    return pl.pallas_call(
        _readout_body,
        grid=(NP // RB,),
        in_specs=[
            pl.BlockSpec((RB, H), lambda b: (b, 0)),
            pl.BlockSpec((1, RB), lambda b: (0, b)),
            pl.BlockSpec((H, NGP), lambda b: (0, 0)),
            pl.BlockSpec((1, NGP), lambda b: (0, 0)),
            pl.BlockSpec((NGP, 128), lambda b: (0, 0)),
            pl.BlockSpec((1, 1), lambda b: (0, 0), memory_space=pltpu.SMEM),
        ],
        out_specs=pl.BlockSpec((128, 1), lambda b: (0, 0)),
        out_shape=jax.ShapeDtypeStruct((128, 1), jnp.float32),
    )(h, batchT, ow1, ob1, ow2p, ob2)


def kernel(z, pos, batch, emb, mlp_w1, mlp_b1, mlp_w2, mlp_b2, conv_w1,
           conv_w2, conv_b2, lin_w, lin_b, out_w1, out_b1, out_w2, out_b2):
    zi = z.astype(jnp.int32)
    bi = batch.astype(jnp.int32)
    pos = pos.astype(jnp.float32)

    gids = jnp.arange(NGRAPH, dtype=jnp.int32)
    seg_start = jnp.searchsorted(bi, gids, side="left").astype(jnp.int32)
    seg_end = jnp.searchsorted(bi, gids, side="right").astype(jnp.int32)
    row_start = seg_start[bi]
    row_end = seg_end[bi]
    zpad = jnp.zeros((NP - N,), jnp.int32)
    row_start_p = jnp.concatenate([row_start, zpad])
    row_end_p = jnp.concatenate([row_end, zpad])

    pos_p = jnp.concatenate([pos, jnp.zeros((NP - N, 3), jnp.float32)], 0)
    sq = jnp.sum(pos_p * pos_p, axis=1, keepdims=True)
    zeros4 = jnp.zeros((NP, 4), jnp.float32)
    colfeat = jnp.concatenate([pos_p, sq, zeros4], axis=1)
    rowT = jnp.concatenate([pos_p.T, sq.T, zeros4.T], axis=0)

    blo = row_start_p.reshape(NB, BR)[:, 0]
    bhi = jnp.max(row_end_p.reshape(NB, BR), axis=1)
    cs_tile = blo // BC
    n_tiles = (bhi + BC - 1) // BC - cs_tile

    rsT = row_start_p[None, :]
    reT = row_end_p[None, :]
    ridT = jnp.arange(NP, dtype=jnp.int32)[None, :]

    srcT, wT, cT = _neighbor_search(cs_tile, n_tiles, rowT, colfeat,
                                    rsT, reT, ridT)
    src = srcT.reshape(NB, K, BR).transpose(0, 2, 1).reshape(E)
    src2d = src.reshape(E // CH, CH)
    wcol = wT.reshape(NB, K, BR).transpose(0, 2, 1).reshape(E, 1)
    ccol = cT.reshape(NB, K, BR).transpose(0, 2, 1).reshape(E, 1)

    z_pad = jnp.concatenate([zi, jnp.zeros((NP - N,), jnp.int32)])[:, None]
    emb_pad = jnp.concatenate(
        [emb.astype(jnp.float32), jnp.zeros((28, H), jnp.float32)], 0)
    w1p = jnp.concatenate(
        [mlp_w1, jnp.zeros((NI, NGP - NG, NF), mlp_w1.dtype)], 1)
    ow1 = jnp.concatenate(
        [out_w1, jnp.zeros((H, NGP - H // 2), out_w1.dtype)], 1)
    ob1 = jnp.concatenate(
        [out_b1, jnp.zeros((NGP - H // 2,), out_b1.dtype)], 0)[None, :]
    ow2p = jnp.zeros((NGP, 128), jnp.float32).at[: H // 2, 0].set(out_w2[:, 0])
    ob2 = out_b2.reshape(1, 1)
    batchT = jnp.concatenate(
        [bi, jnp.full((NP - N,), 127, jnp.int32)])[None, :]

    h, xh = _embed(z_pad, emb_pad, conv_w1[0])

    zeros_hh = jnp.zeros((H, H), jnp.float32)
    for i in range(NI):
        gath = _sc_gather(xh, src2d)
        cw1n = conv_w1[i + 1] if i + 1 < NI else zeros_hh
        h, xh = _layer(wcol, ccol, gath, h,
                       w1p[i], mlp_b1[i][None, :], mlp_w2[i],
                       mlp_b2[i][None, :], conv_w2[i], conv_b2[i][None, :],
                       lin_w[i], lin_b[i][None, :], cw1n)

    res = _readout(h, batchT, ow1, ob1, ow2p, ob2)
    return res[:NGRAPH]


# speedup vs baseline: 9.3107x; 1.0104x over previous
import functools
import math

import jax
import jax.numpy as jnp
from jax import lax
from jax.experimental import pallas as pl
from jax.experimental.pallas import tpu as pltpu
from jax.experimental.pallas import tpu_sc as plsc

N = 10000
H = 128
NF = 128
NG = 50
NGP = 64
NI = 6
K = 32
CUTOFF = 10.0
NGRAPH = 100

BR = 128
BC = 128
NP = 10240
NB = NP // BR
E = NP * K

LOG2 = math.log(2.0)
DELTA = CUTOFF / (NG - 1)
GCOEFF = -0.5 / (DELTA * DELTA)

NC_SC = 2
NS_SC = 16
NWORK = NC_SC * NS_SC
EPW = E // NWORK
CH = 128
NCH = EPW // CH


def _ssp(x):
    return jnp.maximum(x, 0.0) + jnp.log1p(jnp.exp(-jnp.abs(x))) - LOG2


def _nbr_body(cs_ref, nt_ref, rowt_ref, colf_ref, rs_ref, re_ref, rid_ref,
              src_ref, w_ref, c_ref):
    b = pl.program_id(0)
    n_tiles = nt_ref[b]
    cs_tile = cs_ref[b]

    rowt = rowt_ref[...]
    subl = lax.broadcasted_iota(jnp.int32, (8, 1), 0)
    rowp = jnp.where(subl < 3, rowt, 0.0)
    rx = rowt[0:1, :]
    ry = rowt[1:2, :]
    rz = rowt[2:3, :]
    sqr = rowt[3:4, :]
    rs = rs_ref[...]
    re = re_ref[...]
    rid = rid_ref[...]

    inf = jnp.float32(jnp.inf)
    best_d0 = jnp.full((K, BR), inf, jnp.float32)
    best_w0 = jnp.full((K, BR), inf, jnp.float32)
    best_i0 = jnp.zeros((K, BR), jnp.int32)
    krow = lax.broadcasted_iota(jnp.int32, (K, BR), 0)

    def tile_body(ct, carry):
        bd, bw, bi = carry
        c0 = (cs_tile + ct) * BC
        colf = colf_ref[pl.ds(c0, BC), :]
        sqc = colf[:, 3:4]
        lane8 = lax.broadcasted_iota(jnp.int32, (1, 8), 1)
        colp = jnp.where(lane8 < 3, colf, 0.0)
        dot = lax.dot_general(colp, rowp, (((1,), (0,)), ((), ())),
                              preferred_element_type=jnp.float32)
        d2 = (sqc + sqr) - 2.0 * dot
        xc = colf[:, 0:1]
        yc = colf[:, 1:2]
        zc = colf[:, 2:3]
        dotf = (xc * rx + yc * ry) + zc * rz
        d2w = jnp.maximum((sqc + sqr) - 2.0 * dotf, 0.0)
        cg = c0 + lax.broadcasted_iota(jnp.int32, (BC, 1), 0)
        m = (cg >= rs) & (cg < re) & (cg != rid)
        d2m = jnp.where(m, jnp.maximum(d2, 0.0), inf)
        cgb = jnp.broadcast_to(cg, (BC, BR))

        comb_d = jnp.concatenate([bd, d2m], axis=0)
        comb_w = jnp.concatenate([bw, jnp.broadcast_to(d2w, (BC, BR))], axis=0)
        comb_i = jnp.concatenate([bi, cgb], axis=0)
        nd = jnp.full((K, BR), inf, jnp.float32)
        nw = jnp.full((K, BR), inf, jnp.float32)
        ni = jnp.zeros((K, BR), jnp.int32)
        big = jnp.int32(1 << 30)
        for k in range(K):
            mv = jnp.min(comb_d, axis=0, keepdims=True)
            ismin = comb_d == mv
            iv = jnp.min(jnp.where(ismin, comb_i, big), axis=0,
                         keepdims=True)
            sel = ismin & (comb_i == iv)
            vw = jnp.min(jnp.where(sel, comb_w, inf), axis=0,
                         keepdims=True)
            nd = jnp.where(krow == k, mv, nd)
            nw = jnp.where(krow == k, vw, nw)
            ni = jnp.where(krow == k, iv, ni)
            comb_d = jnp.where(sel, inf, comb_d)
        return nd, nw, ni

    best_d, best_w, best_i = lax.fori_loop(
        0, n_tiles, tile_body, (best_d0, best_w0, best_i0))

    valid = best_d < jnp.float32(CUTOFF * CUTOFF)
    ridb = jnp.broadcast_to(rid, (K, BR))
    src = jnp.where(valid, best_i, ridb)
    d2e = jnp.where(valid, best_w, jnp.float32(CUTOFF * CUTOFF))
    wv = jnp.sqrt(d2e)
    cv = 0.5 * (jnp.cos(wv * jnp.float32(math.pi / CUTOFF)) + 1.0)
    src_ref[...] = src
    w_ref[...] = wv
    c_ref[...] = cv


def _neighbor_search(cs_tile, n_tiles, rowT, colfeat, rsT, reT, ridT):
    grid_spec = pltpu.PrefetchScalarGridSpec(
        num_scalar_prefetch=2,
        grid=(NB,),
        in_specs=[
            pl.BlockSpec((8, BR), lambda b, cs, nt: (0, b)),
            pl.BlockSpec((NP, 8), lambda b, cs, nt: (0, 0)),
            pl.BlockSpec((1, BR), lambda b, cs, nt: (0, b)),
            pl.BlockSpec((1, BR), lambda b, cs, nt: (0, b)),
            pl.BlockSpec((1, BR), lambda b, cs, nt: (0, b)),
        ],
        out_specs=[
            pl.BlockSpec((K, BR), lambda b, cs, nt: (b, 0)),
            pl.BlockSpec((K, BR), lambda b, cs, nt: (b, 0)),
            pl.BlockSpec((K, BR), lambda b, cs, nt: (b, 0)),
        ],
    )
    return pl.pallas_call(
        _nbr_body,
        grid_spec=grid_spec,
        out_shape=[
            jax.ShapeDtypeStruct((NB * K, BR), jnp.int32),
            jax.ShapeDtypeStruct((NB * K, BR), jnp.float32),
            jax.ShapeDtypeStruct((NB * K, BR), jnp.float32),
        ],
    )(cs_tile, n_tiles, rowT, colfeat, rsT, reT, ridT)


def _embed_body(z_ref, emb_ref, cw1_ref, h_ref, xh_ref):
    zt = z_ref[...]
    lane = lax.broadcasted_iota(jnp.int32, (BR, 128), 1)
    onehot = (lane == zt).astype(jnp.float32)
    h = jnp.dot(onehot, emb_ref[...], preferred_element_type=jnp.float32,
                precision=lax.Precision.HIGHEST)
    h_ref[...] = h
    xh_ref[...] = jnp.dot(h, cw1_ref[...], preferred_element_type=jnp.float32)


def _embed(z_pad, emb_pad, cw1_0):
    return pl.pallas_call(
        _embed_body,
        grid=(NB,),
        in_specs=[
            pl.BlockSpec((BR, 1), lambda b: (b, 0)),
            pl.BlockSpec((128, H), lambda b: (0, 0)),
            pl.BlockSpec((H, H), lambda b: (0, 0)),
        ],
        out_specs=[
            pl.BlockSpec((BR, H), lambda b: (b, 0)),
            pl.BlockSpec((BR, H), lambda b: (b, 0)),
        ],
        out_shape=[
            jax.ShapeDtypeStruct((NP, H), jnp.float32),
            jax.ShapeDtypeStruct((NP, H), jnp.float32),
        ],
    )(z_pad, emb_pad, cw1_0)


NBUF = 5
NROUNDS = NCH // NBUF


def _sc_gather(xh, src2d):
    mesh = plsc.VectorSubcoreMesh(core_axis_name="c", subcore_axis_name="s")

    @functools.partial(
        pl.kernel,
        mesh=mesh,
        out_type=jax.ShapeDtypeStruct((E, H), jnp.float32),
        scratch_types=[
            pltpu.VMEM((NCH, CH), jnp.int32),
            pltpu.VMEM((NBUF, CH, H), jnp.float32),
        ]
        + [pltpu.SemaphoreType.DMA] * (2 * NBUF),
    )
    def gather_k(xh_hbm, src_hbm, out_hbm, idx_v, rows_v, *sems):
        gsem = sems[:NBUF]
        wsem = sems[NBUF:]
        wid = lax.axis_index("s") * NC_SC + lax.axis_index("c")
        ebase = wid * EPW
        pltpu.sync_copy(src_hbm.at[pl.ds(wid * NCH, NCH)], idx_v)

        def gstart(ch, b):
            pltpu.async_copy(xh_hbm.at[idx_v.at[ch]], rows_v.at[b], gsem[b])

        def gwait(b):
            pltpu.make_async_copy(xh_hbm.at[idx_v.at[0]], rows_v.at[b],
                                  gsem[b]).wait()

        def wstart(ch, b):
            pltpu.async_copy(rows_v.at[b],
                             out_hbm.at[pl.ds(ebase + ch * CH, CH)], wsem[b])

        def wwait(b):
            pltpu.make_async_copy(rows_v.at[b],
                                  out_hbm.at[pl.ds(ebase, CH)], wsem[b]).wait()

        for b in range(NBUF):
            gstart(b, b)

        def round_body(g, carry):
            for b in range(NBUF):
                gwait(b)
                wstart(g * NBUF + b, b)
            nxt = g + 1

            @pl.when(nxt < NROUNDS)
            def _():
                for b in range(NBUF):
                    wwait(b)
                    gstart(nxt * NBUF + b, b)

            return carry

        lax.fori_loop(0, NROUNDS, round_body, 0)
        for b in range(NBUF):
            wwait(b)

    return gather_k(xh, src2d)


def _layer_body(w_ref, c_ref, g_ref, h_ref, w1_ref, b1_ref, w2_ref, b2_ref,
                cw2_ref, cb2_ref, lw_ref, lb_ref, cw1n_ref,
                hn_ref, xhn_ref):
    wt = w_ref[...]
    offs = lax.broadcasted_iota(jnp.int32, (1, NGP), 1).astype(
        jnp.float32) * jnp.float32(DELTA)
    d = wt - offs
    attr = jnp.exp(jnp.float32(GCOEFF) * d * d)
    z1 = jnp.dot(attr, w1_ref[...], preferred_element_type=jnp.float32)
    z1 = _ssp(z1 + b1_ref[...])
    z2 = jnp.dot(z1, w2_ref[...], preferred_element_type=jnp.float32)
    z2 = (z2 + b2_ref[...]) * c_ref[...]
    msg = g_ref[...] * z2
    m3 = msg.reshape(BR, K, H)
    agg = jnp.sum(m3, axis=1)
    t = jnp.dot(agg, cw2_ref[...], preferred_element_type=jnp.float32)
    t = _ssp(t + cb2_ref[...])
    u = jnp.dot(t, lw_ref[...], preferred_element_type=jnp.float32)
    hn = h_ref[...] + u + lb_ref[...]
    hn_ref[...] = hn
    xhn_ref[...] = jnp.dot(hn, cw1n_ref[...],
                           preferred_element_type=jnp.float32)


def _layer(wcol, ccol, gath, h, w1p, b1, w2, b2, cw2, cb2, lw, lb, cw1n):
    EB = BR * K
    return pl.pallas_call(
        _layer_body,
        grid=(NB,),
        in_specs=[
            pl.BlockSpec((EB, 1), lambda b: (b, 0)),
            pl.BlockSpec((EB, 1), lambda b: (b, 0)),
            pl.BlockSpec((EB, H), lambda b: (b, 0)),
            pl.BlockSpec((BR, H), lambda b: (b, 0)),
            pl.BlockSpec((NGP, NF), lambda b: (0, 0)),
            pl.BlockSpec((1, NF), lambda b: (0, 0)),
            pl.BlockSpec((NF, NF), lambda b: (0, 0)),
            pl.BlockSpec((1, NF), lambda b: (0, 0)),
            pl.BlockSpec((NF, H), lambda b: (0, 0)),
            pl.BlockSpec((1, H), lambda b: (0, 0)),
            pl.BlockSpec((H, H), lambda b: (0, 0)),
            pl.BlockSpec((1, H), lambda b: (0, 0)),
            pl.BlockSpec((H, H), lambda b: (0, 0)),
        ],
        out_specs=[
            pl.BlockSpec((BR, H), lambda b: (b, 0)),
            pl.BlockSpec((BR, H), lambda b: (b, 0)),
        ],
        out_shape=[
            jax.ShapeDtypeStruct((NP, H), jnp.float32),
            jax.ShapeDtypeStruct((NP, H), jnp.float32),
        ],
    )(wcol, ccol, gath, h, w1p, b1, w2, b2, cw2, cb2, lw, lb, cw1n)


def _readout_body(h_ref, bt_ref, ow1_ref, ob1_ref, ow2_ref, ob2_ref, out_ref):
    b = pl.program_id(0)
    h = h_ref[...]
    t = _ssp(jnp.dot(h, ow1_ref[...], preferred_element_type=jnp.float32)
             + ob1_ref[...])
    y = jnp.dot(t, ow2_ref[...], preferred_element_type=jnp.float32)
    y = y[:, 0:1] + ob2_ref[0, 0]
    gi = lax.broadcasted_iota(jnp.int32, (128, 1), 0)
    mask = (gi == bt_ref[...]).astype(jnp.float32)
    contrib = jnp.dot(mask, y, preferred_element_type=jnp.float32,
                      precision=lax.Precision.HIGHEST)

    @pl.when(b == 0)
    def _():
        out_ref[...] = jnp.zeros_like(out_ref)

    out_ref[...] += contrib


def _readout(h, batchT, ow1, ob1, ow2p, ob2):
    RB = 512
    return pl.pallas_call(
        _readout_body,
        grid=(NP // RB,),
        in_specs=[
            pl.BlockSpec((RB, H), lambda b: (b, 0)),
            pl.BlockSpec((1, RB), lambda b: (0, b)),
            pl.BlockSpec((H, NGP), lambda b: (0, 0)),
            pl.BlockSpec((1, NGP), lambda b: (0, 0)),
            pl.BlockSpec((NGP, 128), lambda b: (0, 0)),
            pl.BlockSpec((1, 1), lambda b: (0, 0), memory_space=pltpu.SMEM),
        ],
        out_specs=pl.BlockSpec((128, 1), lambda b: (0, 0)),
        out_shape=jax.ShapeDtypeStruct((128, 1), jnp.float32),
    )(h, batchT, ow1, ob1, ow2p, ob2)


def kernel(z, pos, batch, emb, mlp_w1, mlp_b1, mlp_w2, mlp_b2, conv_w1,
           conv_w2, conv_b2, lin_w, lin_b, out_w1, out_b1, out_w2, out_b2):
    zi = z.astype(jnp.int32)
    bi = batch.astype(jnp.int32)
    pos = pos.astype(jnp.float32)

    gids = jnp.arange(NGRAPH, dtype=jnp.int32)
    seg_start = jnp.searchsorted(bi, gids, side="left").astype(jnp.int32)
    seg_end = jnp.searchsorted(bi, gids, side="right").astype(jnp.int32)
    row_start = seg_start[bi]
    row_end = seg_end[bi]
    zpad = jnp.zeros((NP - N,), jnp.int32)
    row_start_p = jnp.concatenate([row_start, zpad])
    row_end_p = jnp.concatenate([row_end, zpad])

    pos_p = jnp.concatenate([pos, jnp.zeros((NP - N, 3), jnp.float32)], 0)
    sq = jnp.sum(pos_p * pos_p, axis=1, keepdims=True)
    zeros4 = jnp.zeros((NP, 4), jnp.float32)
    colfeat = jnp.concatenate([pos_p, sq, zeros4], axis=1)
    rowT = jnp.concatenate([pos_p.T, sq.T, zeros4.T], axis=0)

    blo = row_start_p.reshape(NB, BR)[:, 0]
    bhi = jnp.max(row_end_p.reshape(NB, BR), axis=1)
    cs_tile = blo // BC
    n_tiles = (bhi + BC - 1) // BC - cs_tile

    rsT = row_start_p[None, :]
    reT = row_end_p[None, :]
    ridT = jnp.arange(NP, dtype=jnp.int32)[None, :]

    srcT, wT, cT = _neighbor_search(cs_tile, n_tiles, rowT, colfeat,
                                    rsT, reT, ridT)
    src = srcT.reshape(NB, K, BR).transpose(0, 2, 1).reshape(E)
    src2d = src.reshape(E // CH, CH)
    wcol = wT.reshape(NB, K, BR).transpose(0, 2, 1).reshape(E, 1)
    ccol = cT.reshape(NB, K, BR).transpose(0, 2, 1).reshape(E, 1)

    z_pad = jnp.concatenate([zi, jnp.zeros((NP - N,), jnp.int32)])[:, None]
    emb_pad = jnp.concatenate(
        [emb.astype(jnp.float32), jnp.zeros((28, H), jnp.float32)], 0)
    w1p = jnp.concatenate(
        [mlp_w1, jnp.zeros((NI, NGP - NG, NF), mlp_w1.dtype)], 1)
    ow1 = jnp.concatenate(
        [out_w1, jnp.zeros((H, NGP - H // 2), out_w1.dtype)], 1)
    ob1 = jnp.concatenate(
        [out_b1, jnp.zeros((NGP - H // 2,), out_b1.dtype)], 0)[None, :]
    ow2p = jnp.zeros((NGP, 128), jnp.float32).at[: H // 2, 0].set(out_w2[:, 0])
    ob2 = out_b2.reshape(1, 1)
    batchT = jnp.concatenate(
        [bi, jnp.full((NP - N,), 127, jnp.int32)])[None, :]

    h, xh = _embed(z_pad, emb_pad, conv_w1[0])

    zeros_hh = jnp.zeros((H, H), jnp.float32)
    for i in range(NI):
        gath = _sc_gather(xh, src2d)
        cw1n = conv_w1[i + 1] if i + 1 < NI else zeros_hh
        h, xh = _layer(wcol, ccol, gath, h,
                       w1p[i], mlp_b1[i][None, :], mlp_w2[i],
                       mlp_b2[i][None, :], conv_w2[i], conv_b2[i][None, :],
                       lin_w[i], lin_b[i][None, :], cw1n)

    res = _readout(h, batchT, ow1, ob1, ow2p, ob2)
    return res[:NGRAPH]
